# baseline (speedup 1.0000x reference)
import functools

import jax
import jax.numpy as jnp
from jax import lax
from jax.experimental import pallas as pl
from jax.experimental.pallas import tpu as pltpu
from jax.experimental.pallas import tpu_sc as plsc

VOCAB = 1000000
EMBED_DIM = 64
BATCH = 4096
HIST = 50
TOT = BATCH * HIST

NC = 2
NS = 16
NW = NC * NS

CHUNK = 128
PER_W = TOT // NW
N_CH = PER_W // CHUNK
NBUF = 2
N_GRP = N_CH // NBUF


def _build():
  mesh = plsc.VectorSubcoreMesh(core_axis_name="c", subcore_axis_name="s")

  @functools.partial(
      pl.kernel,
      mesh=mesh,
      compiler_params=pltpu.CompilerParams(use_tc_tiling_on_sc=False),
      out_type=jax.ShapeDtypeStruct((TOT, EMBED_DIM), jnp.float32),
      scratch_types=[
          pltpu.VMEM((N_CH, CHUNK), jnp.int32),
          pltpu.VMEM((CHUNK, EMBED_DIM), jnp.float32),
          pltpu.VMEM((CHUNK, EMBED_DIM), jnp.float32),
          pltpu.SemaphoreType.DMA,
          pltpu.SemaphoreType.DMA,
      ],
  )
  def emb_kernel(idx_hbm, table_hbm, out_hbm, idx_v, buf0, buf1, sem0, sem1):
    bufs = (buf0, buf1)
    sems = (sem0, sem1)
    wid = lax.axis_index("s") * NC + lax.axis_index("c")
    base_row = wid * PER_W
    pltpu.sync_copy(idx_hbm.at[wid], idx_v)
    for b in range(NBUF):
      pltpu.async_copy(table_hbm.at[idx_v.at[b]], bufs[b], sems[b])

    def body(g, _):
      for b in range(NBUF):
        c = g * NBUF + b
        pltpu.make_async_copy(table_hbm.at[idx_v.at[c]], bufs[b],
                              sems[b]).wait()
        pltpu.sync_copy(bufs[b], out_hbm.at[pl.ds(base_row + c * CHUNK,
                                                  CHUNK)])
        nxt = c + NBUF

        @pl.when(nxt < N_CH)
        def _():
          pltpu.async_copy(table_hbm.at[idx_v.at[nxt]], bufs[b], sems[b])

      return ()

    lax.fori_loop(0, N_GRP, body, (), unroll=False)

  return emb_kernel


_emb_kernel = _build()


@jax.jit
def kernel(inputs, embeddings):
  idx = inputs.reshape(NW, N_CH, CHUNK)
  out = _emb_kernel(idx, embeddings)
  return out.reshape(BATCH, HIST, EMBED_DIM)


# speedup vs baseline: 1.0094x; 1.0094x over previous
import functools

import jax
import jax.numpy as jnp
from jax import lax
from jax.experimental import pallas as pl
from jax.experimental.pallas import tpu as pltpu
from jax.experimental.pallas import tpu_sc as plsc

VOCAB = 1000000
EMBED_DIM = 64
BATCH = 4096
HIST = 50
TOT = BATCH * HIST

NC = 2
NS = 16
NW = NC * NS

CHUNK = 128
PER_W = TOT // NW
N_CH = PER_W // CHUNK
CPB = 5
BROWS = CPB * CHUNK
NBAT = N_CH // CPB


def _build():
  mesh = plsc.VectorSubcoreMesh(core_axis_name="c", subcore_axis_name="s")

  @functools.partial(
      pl.kernel,
      mesh=mesh,
      compiler_params=pltpu.CompilerParams(use_tc_tiling_on_sc=False),
      out_type=jax.ShapeDtypeStruct((TOT, EMBED_DIM), jnp.float32),
      scratch_types=[
          pltpu.VMEM((N_CH, CHUNK), jnp.int32),
          pltpu.VMEM((BROWS, EMBED_DIM), jnp.float32),
          pltpu.VMEM((BROWS, EMBED_DIM), jnp.float32),
          pltpu.SemaphoreType.DMA,
          pltpu.SemaphoreType.DMA,
          pltpu.SemaphoreType.DMA,
          pltpu.SemaphoreType.DMA,
      ],
  )
  def emb_kernel(idx_hbm, table_hbm, out_hbm, idx_v, buf0, buf1,
                 gsem0, gsem1, ssem0, ssem1):
    bufs = (buf0, buf1)
    gsems = (gsem0, gsem1)
    ssems = (ssem0, ssem1)
    wid = lax.axis_index("s") * NC + lax.axis_index("c")
    base_row = wid * PER_W
    pltpu.sync_copy(idx_hbm.at[wid], idx_v)

    def fire_gathers(t, b):
      for k in range(CPB):
        pltpu.async_copy(table_hbm.at[idx_v.at[t * CPB + k]],
                         bufs[b].at[pl.ds(k * CHUNK, CHUNK)], gsems[b])

    def drain_gathers(t, b):
      for k in range(CPB):
        pltpu.make_async_copy(table_hbm.at[idx_v.at[t * CPB + k]],
                              bufs[b].at[pl.ds(k * CHUNK, CHUNK)],
                              gsems[b]).wait()

    fire_gathers(0, 0)
    fire_gathers(1, 1)

    def body(g, _):
      for b in range(2):
        t = g * 2 + b
        drain_gathers(t, b)
        pltpu.async_copy(bufs[b], out_hbm.at[pl.ds(base_row + t * BROWS,
                                                   BROWS)], ssems[b])
        nxt = t + 2

        @pl.when(nxt < NBAT)
        def _():
          pltpu.make_async_copy(bufs[b],
                                out_hbm.at[pl.ds(base_row + t * BROWS,
                                                 BROWS)], ssems[b]).wait()
          fire_gathers(nxt, b)

      return ()

    lax.fori_loop(0, NBAT // 2, body, (), unroll=False)
    for b in range(2):
      t = NBAT - 2 + b
      pltpu.make_async_copy(bufs[b],
                            out_hbm.at[pl.ds(base_row + t * BROWS, BROWS)],
                            ssems[b]).wait()

  return emb_kernel


_emb_kernel = _build()


@jax.jit
def kernel(inputs, embeddings):
  idx = inputs.reshape(NW, N_CH, CHUNK)
  out = _emb_kernel(idx, embeddings)
  return out.reshape(BATCH, HIST, EMBED_DIM)
